# baseline (speedup 1.0000x reference)
import functools

import jax
import jax.numpy as jnp
from jax import lax
from jax.experimental import pallas as pl
from jax.experimental.pallas import tpu as pltpu
from jax.experimental.pallas import tpu_sc as plsc

BATCH = 16384
MF_DIM = 32
MLP_HALF = 32


def _sc_gather(user_idx, item_idx, user_emb_mf, item_emb_mf, user_emb_mlp, item_emb_mlp):
    info = plsc.get_sparse_core_info()
    nc, ns = info.num_cores, info.num_subcores
    nw = nc * ns
    bpw = BATCH // nw

    mesh = plsc.VectorSubcoreMesh(core_axis_name="c", subcore_axis_name="s")
    row_t = jax.ShapeDtypeStruct((BATCH, MF_DIM), jnp.float32)

    @functools.partial(
        pl.kernel,
        mesh=mesh,
        out_type=[row_t, row_t, row_t, row_t],
        scratch_types=[
            pltpu.VMEM((bpw,), jnp.int32),
            pltpu.VMEM((bpw,), jnp.int32),
            pltpu.SemaphoreType.DMA,
            pltpu.SemaphoreType.DMA,
        ],
    )
    def gather_kernel(uidx_hbm, iidx_hbm, umf_hbm, imf_hbm, umlp_hbm, imlp_hbm,
                      o_umf, o_imf, o_umlp, o_imlp,
                      uidx_v, iidx_v, s0, si):
        wid = lax.axis_index("s") * nc + lax.axis_index("c")
        base = wid * bpw
        ci = pltpu.async_copy(uidx_hbm.at[pl.ds(base, bpw)], uidx_v, si)
        cj = pltpu.async_copy(iidx_hbm.at[pl.ds(base, bpw)], iidx_v, si)
        ci.wait()
        cj.wait()

        def body(g, _):
            uvec = uidx_v[pl.ds(g * 16, 16)]
            ivec = iidx_v[pl.ds(g * 16, 16)]
            for k in range(16):
                u = uvec[k]
                i = ivec[k]
                d = base + g * 16 + k
                pltpu.async_copy(umf_hbm.at[u], o_umf.at[d], s0)
                pltpu.async_copy(imf_hbm.at[i], o_imf.at[d], s0)
                pltpu.async_copy(umlp_hbm.at[u], o_umlp.at[d], s0)
                pltpu.async_copy(imlp_hbm.at[i], o_imlp.at[d], s0)
            return _

        lax.fori_loop(0, bpw // 16, body, 0)
        pltpu.make_async_copy(o_umf.at[pl.ds(base, bpw)], o_umf.at[pl.ds(base, bpw)], s0).wait()
        pltpu.make_async_copy(o_imf.at[pl.ds(base, bpw)], o_imf.at[pl.ds(base, bpw)], s0).wait()
        pltpu.make_async_copy(o_umlp.at[pl.ds(base, bpw)], o_umlp.at[pl.ds(base, bpw)], s0).wait()
        pltpu.make_async_copy(o_imlp.at[pl.ds(base, bpw)], o_imlp.at[pl.ds(base, bpw)], s0).wait()

    return gather_kernel(user_idx, item_idx, user_emb_mf, item_emb_mf,
                         user_emb_mlp, item_emb_mlp)


def _tc_body(umf, imf, umlp, imlp, w1u, w1i, b1, w2, b2, wp1, wp2, bp, out):
    gmf = umf[...] * imf[...]
    h1 = jnp.dot(umlp[...], w1u[...], preferred_element_type=jnp.float32)
    h1 = h1 + jnp.dot(imlp[...], w1i[...], preferred_element_type=jnp.float32)
    h1 = jnp.maximum(h1 + b1[...], 0.0)
    h2 = jnp.dot(h1, w2[...], preferred_element_type=jnp.float32)
    h2 = jnp.maximum(h2 + b2[...], 0.0)
    logits = jnp.sum(gmf * wp1[...], axis=1) + jnp.sum(h2 * wp2[...], axis=1)
    out[...] = jnp.clip(logits + bp[0, 0], -15.0, 15.0)


def _tc_mlp(umf, imf, umlp, imlp, W1, b1, W2, b2, Wp, bp):
    blk = 2048
    grid = BATCH // blk
    w1u = W1[:, :MLP_HALF].T
    w1i = W1[:, MLP_HALF:].T
    b1r = b1.reshape(1, -1)
    w2 = W2.T
    b2r = b2.reshape(1, -1)
    wp1 = Wp[:, :MF_DIM]
    wp2 = Wp[:, MF_DIM:]
    bpr = bp.reshape(1, 1)

    act_spec = pl.BlockSpec((blk, MF_DIM), lambda i: (i, 0))
    full = lambda shape: pl.BlockSpec(shape, lambda i: (0,) * len(shape))
    return pl.pallas_call(
        _tc_body,
        grid=(grid,),
        in_specs=[
            act_spec, act_spec, act_spec, act_spec,
            full((32, 32)), full((32, 32)), full((1, 32)),
            full((32, 16)), full((1, 16)),
            full((1, 32)), full((1, 16)), full((1, 1)),
        ],
        out_specs=pl.BlockSpec((blk,), lambda i: (i,)),
        out_shape=jax.ShapeDtypeStruct((BATCH,), jnp.float32),
    )(umf, imf, umlp, imlp, w1u, w1i, b1r, w2, b2r, wp1, wp2, bpr)


def kernel(user_idx, item_idx, user_emb_mf, item_emb_mf, user_emb_mlp, item_emb_mlp,
           W1, b1, W2, b2, Wp, bp):
    umf, imf, umlp, imlp = _sc_gather(
        user_idx.astype(jnp.int32), item_idx.astype(jnp.int32),
        user_emb_mf, item_emb_mf, user_emb_mlp, item_emb_mlp)
    return _tc_mlp(umf, imf, umlp, imlp, W1, b1, W2, b2, Wp, bp)


# speedup vs baseline: 1.2444x; 1.2444x over previous
import functools

import jax
import jax.numpy as jnp
from jax import lax
from jax.experimental import pallas as pl
from jax.experimental.pallas import tpu as pltpu
from jax.experimental.pallas import tpu_sc as plsc

BATCH = 16384
MF_DIM = 32
MLP_HALF = 32
NUM_ROWS = 1000000
RPS = 8
SLICE_W = MF_DIM * RPS


def _sc_gather(user_idx, item_idx, umf8, imf8, umlp8, imlp8):
    info = plsc.get_sparse_core_info()
    nc, ns = info.num_cores, info.num_subcores
    nw = nc * ns
    bpw = BATCH // nw
    C = 64

    mesh = plsc.VectorSubcoreMesh(core_axis_name="c", subcore_axis_name="s")
    row_t = jax.ShapeDtypeStruct((BATCH, MF_DIM), jnp.float32)

    @functools.partial(
        pl.kernel,
        mesh=mesh,
        compiler_params=pltpu.CompilerParams(needs_layout_passes=False),
        out_type=[row_t, row_t, row_t, row_t],
        scratch_types=[
            pltpu.VMEM((bpw,), jnp.int32),
            pltpu.VMEM((bpw,), jnp.int32),
            pltpu.VMEM((bpw,), jnp.int32),
            pltpu.VMEM((bpw,), jnp.int32),
            pltpu.VMEM((C, SLICE_W), jnp.float32),
            pltpu.VMEM((C, SLICE_W), jnp.float32),
            pltpu.VMEM((C, SLICE_W), jnp.float32),
            pltpu.VMEM((C, SLICE_W), jnp.float32),
            pltpu.VMEM((C, MF_DIM), jnp.float32),
            pltpu.VMEM((C, MF_DIM), jnp.float32),
            pltpu.VMEM((C, MF_DIM), jnp.float32),
            pltpu.VMEM((C, MF_DIM), jnp.float32),
            [pltpu.SemaphoreType.DMA] * 4,
            pltpu.SemaphoreType.DMA,
        ],
    )
    def gather_kernel(uidx_hbm, iidx_hbm, umf_hbm, imf_hbm, umlp_hbm, imlp_hbm,
                      o_umf, o_imf, o_umlp, o_imlp,
                      uidx_v, iidx_v, u3_v, i3_v,
                      g_umf, g_imf, g_umlp, g_imlp,
                      st_umf, st_imf, st_umlp, st_imlp,
                      sems, si):
        wid = lax.axis_index("s") * nc + lax.axis_index("c")
        base = wid * bpw
        ci = pltpu.async_copy(uidx_hbm.at[pl.ds(base, bpw)], uidx_v, si)
        cj = pltpu.async_copy(iidx_hbm.at[pl.ds(base, bpw)], iidx_v, si)
        ci.wait()
        cj.wait()

        def shift_body(g, _):
            sl = pl.ds(g * 16, 16)
            u3_v[sl] = lax.shift_right_logical(uidx_v[sl], 3)
            i3_v[sl] = lax.shift_right_logical(iidx_v[sl], 3)
            return _

        lax.fori_loop(0, bpw // 16, shift_body, 0)

        tables = (umf_hbm, imf_hbm, umlp_hbm, imlp_hbm)
        gbufs = (g_umf, g_imf, g_umlp, g_imlp)
        stages = (st_umf, st_imf, st_umlp, st_imlp)
        outs = (o_umf, o_imf, o_umlp, o_imlp)
        lanes = lax.iota(jnp.int32, 16)

        def chunk_body(ch, _):
            off = ch * C
            uslice = u3_v.at[pl.ds(off, C)]
            islice = i3_v.at[pl.ds(off, C)]
            idx_slices = (uslice, islice, uslice, islice)
            copies = []
            for t in range(4):
                copies.append(pltpu.async_copy(
                    tables[t].at[idx_slices[t]], gbufs[t], sems[t]))
            for t in range(4):
                copies[t].wait()
            for g in range(C // 16):
                su_vec = lax.bitwise_and(uidx_v[pl.ds(off + g * 16, 16)], 7) * MF_DIM
                si_vec = lax.bitwise_and(iidx_v[pl.ds(off + g * 16, 16)], 7) * MF_DIM
                for k in range(16):
                    c = g * 16 + k
                    c_vec = jnp.full((16,), c, dtype=jnp.int32)
                    svec = (su_vec[k] + lanes, si_vec[k] + lanes)
                    for t in range(4):
                        sv = svec[t % 2]
                        for half in (0, 16):
                            vals = plsc.load_gather(
                                gbufs[t], [c_vec, sv + half])
                            stages[t][c, pl.ds(half, 16)] = vals
            dst = pl.ds(base + off, C)
            for t in range(4):
                pltpu.sync_copy(stages[t], outs[t].at[dst])
            return _

        lax.fori_loop(0, bpw // C, chunk_body, 0)

    return gather_kernel(user_idx, item_idx, umf8, imf8, umlp8, imlp8)


def _tc_body(umf, imf, umlp, imlp, w1u, w1i, b1, w2, b2, wp1, wp2, bp, out):
    gmf = umf[...] * imf[...]
    h1 = jnp.dot(umlp[...], w1u[...], preferred_element_type=jnp.float32)
    h1 = h1 + jnp.dot(imlp[...], w1i[...], preferred_element_type=jnp.float32)
    h1 = jnp.maximum(h1 + b1[...], 0.0)
    h2 = jnp.dot(h1, w2[...], preferred_element_type=jnp.float32)
    h2 = jnp.maximum(h2 + b2[...], 0.0)
    logits = jnp.sum(gmf * wp1[...], axis=1) + jnp.sum(h2 * wp2[...], axis=1)
    out[...] = jnp.clip(logits + bp[0, 0], -15.0, 15.0)


def _tc_mlp(umf, imf, umlp, imlp, W1, b1, W2, b2, Wp, bp):
    blk = 2048
    grid = BATCH // blk
    w1u = W1[:, :MLP_HALF].T
    w1i = W1[:, MLP_HALF:].T
    b1r = b1.reshape(1, -1)
    w2 = W2.T
    b2r = b2.reshape(1, -1)
    wp1 = Wp[:, :MF_DIM]
    wp2 = Wp[:, MF_DIM:]
    bpr = bp.reshape(1, 1)

    act_spec = pl.BlockSpec((blk, MF_DIM), lambda i: (i, 0))
    full = lambda shape: pl.BlockSpec(shape, lambda i: (0,) * len(shape))
    return pl.pallas_call(
        _tc_body,
        grid=(grid,),
        in_specs=[
            act_spec, act_spec, act_spec, act_spec,
            full((32, 32)), full((32, 32)), full((1, 32)),
            full((32, 16)), full((1, 16)),
            full((1, 32)), full((1, 16)), full((1, 1)),
        ],
        out_specs=pl.BlockSpec((blk,), lambda i: (i,)),
        out_shape=jax.ShapeDtypeStruct((BATCH,), jnp.float32),
    )(umf, imf, umlp, imlp, w1u, w1i, b1r, w2, b2r, wp1, wp2, bpr)


def kernel(user_idx, item_idx, user_emb_mf, item_emb_mf, user_emb_mlp, item_emb_mlp,
           W1, b1, W2, b2, Wp, bp):
    n8 = NUM_ROWS // RPS
    umf, imf, umlp, imlp = _sc_gather(
        user_idx.astype(jnp.int32), item_idx.astype(jnp.int32),
        user_emb_mf.reshape(n8, SLICE_W), item_emb_mf.reshape(n8, SLICE_W),
        user_emb_mlp.reshape(n8, SLICE_W), item_emb_mlp.reshape(n8, SLICE_W))
    return _tc_mlp(umf, imf, umlp, imlp, W1, b1, W2, b2, Wp, bp)


# speedup vs baseline: 1.8204x; 1.4629x over previous
import functools

import jax
import jax.numpy as jnp
from jax import lax
from jax.experimental import pallas as pl
from jax.experimental.pallas import tpu as pltpu
from jax.experimental.pallas import tpu_sc as plsc

BATCH = 16384
MF_DIM = 32
MLP_HALF = 32


def _sc_gather(user_idx, item_idx, user_emb_mf, item_emb_mf, user_emb_mlp, item_emb_mlp):
    info = plsc.get_sparse_core_info()
    nc, ns = info.num_cores, info.num_subcores
    nw = nc * ns
    bpw = BATCH // nw
    C = 128

    mesh = plsc.VectorSubcoreMesh(core_axis_name="c", subcore_axis_name="s")
    row_t = jax.ShapeDtypeStruct((BATCH, MF_DIM), jnp.float32)

    @functools.partial(
        pl.kernel,
        mesh=mesh,
        out_type=[row_t, row_t, row_t, row_t],
        scratch_types=[
            pltpu.VMEM((bpw,), jnp.int32),
            pltpu.VMEM((bpw,), jnp.int32),
            pltpu.VMEM((C, MF_DIM), jnp.float32),
            pltpu.VMEM((C, MF_DIM), jnp.float32),
            pltpu.VMEM((C, MF_DIM), jnp.float32),
            pltpu.VMEM((C, MF_DIM), jnp.float32),
            [pltpu.SemaphoreType.DMA] * 4,
            pltpu.SemaphoreType.DMA,
        ],
    )
    def gather_kernel(uidx_hbm, iidx_hbm, umf_hbm, imf_hbm, umlp_hbm, imlp_hbm,
                      o_umf, o_imf, o_umlp, o_imlp,
                      uidx_v, iidx_v,
                      st_umf, st_imf, st_umlp, st_imlp,
                      sems, si):
        wid = lax.axis_index("s") * nc + lax.axis_index("c")
        base = wid * bpw
        ci = pltpu.async_copy(uidx_hbm.at[pl.ds(base, bpw)], uidx_v, si)
        cj = pltpu.async_copy(iidx_hbm.at[pl.ds(base, bpw)], iidx_v, si)
        ci.wait()
        cj.wait()
        tables = (umf_hbm, imf_hbm, umlp_hbm, imlp_hbm)
        stages = (st_umf, st_imf, st_umlp, st_imlp)
        outs = (o_umf, o_imf, o_umlp, o_imlp)

        def chunk_body(ch, _):
            off = ch * C
            for g in range(C // 16):
                uvec = uidx_v[pl.ds(off + g * 16, 16)]
                ivec = iidx_v[pl.ds(off + g * 16, 16)]
                for k in range(16):
                    c = g * 16 + k
                    u = uvec[k]
                    i = ivec[k]
                    idxs = (u, i, u, i)
                    for t in range(4):
                        pltpu.async_copy(tables[t].at[idxs[t]],
                                         stages[t].at[c], sems[t])
            dst = pl.ds(base + off, C)
            for t in range(4):
                pltpu.make_async_copy(outs[t].at[dst], stages[t], sems[t]).wait()
            for t in range(4):
                pltpu.sync_copy(stages[t], outs[t].at[dst])
            return _

        lax.fori_loop(0, bpw // C, chunk_body, 0)

    return gather_kernel(user_idx, item_idx, user_emb_mf, item_emb_mf,
                         user_emb_mlp, item_emb_mlp)


def _tc_body(umf, imf, umlp, imlp, w1u, w1i, b1, w2, b2, wp1, wp2, bp, out):
    gmf = umf[...] * imf[...]
    h1 = jnp.dot(umlp[...], w1u[...], preferred_element_type=jnp.float32)
    h1 = h1 + jnp.dot(imlp[...], w1i[...], preferred_element_type=jnp.float32)
    h1 = jnp.maximum(h1 + b1[...], 0.0)
    h2 = jnp.dot(h1, w2[...], preferred_element_type=jnp.float32)
    h2 = jnp.maximum(h2 + b2[...], 0.0)
    logits = jnp.sum(gmf * wp1[...], axis=1) + jnp.sum(h2 * wp2[...], axis=1)
    out[...] = jnp.clip(logits + bp[0, 0], -15.0, 15.0)


def _tc_mlp(umf, imf, umlp, imlp, W1, b1, W2, b2, Wp, bp):
    blk = 2048
    grid = BATCH // blk
    w1u = W1[:, :MLP_HALF].T
    w1i = W1[:, MLP_HALF:].T
    b1r = b1.reshape(1, -1)
    w2 = W2.T
    b2r = b2.reshape(1, -1)
    wp1 = Wp[:, :MF_DIM]
    wp2 = Wp[:, MF_DIM:]
    bpr = bp.reshape(1, 1)

    act_spec = pl.BlockSpec((blk, MF_DIM), lambda i: (i, 0))
    full = lambda shape: pl.BlockSpec(shape, lambda i: (0,) * len(shape))
    return pl.pallas_call(
        _tc_body,
        grid=(grid,),
        in_specs=[
            act_spec, act_spec, act_spec, act_spec,
            full((32, 32)), full((32, 32)), full((1, 32)),
            full((32, 16)), full((1, 16)),
            full((1, 32)), full((1, 16)), full((1, 1)),
        ],
        out_specs=pl.BlockSpec((blk,), lambda i: (i,)),
        out_shape=jax.ShapeDtypeStruct((BATCH,), jnp.float32),
    )(umf, imf, umlp, imlp, w1u, w1i, b1r, w2, b2r, wp1, wp2, bpr)


def kernel(user_idx, item_idx, user_emb_mf, item_emb_mf, user_emb_mlp, item_emb_mlp,
           W1, b1, W2, b2, Wp, bp):
    umf, imf, umlp, imlp = _sc_gather(
        user_idx.astype(jnp.int32), item_idx.astype(jnp.int32),
        user_emb_mf, item_emb_mf, user_emb_mlp, item_emb_mlp)
    return _tc_mlp(umf, imf, umlp, imlp, W1, b1, W2, b2, Wp, bp)
